# baseline (speedup 1.0000x reference)
import functools

import jax
import jax.numpy as jnp
from jax import lax
from jax.experimental import pallas as pl
from jax.experimental.pallas import tpu as pltpu
from jax.experimental.pallas import tpu_sc as plsc

B, D, S = 4, 1024, 8192
NW = 32
N_DG = 16
D_PER_W = D // N_DG
PE_DW = 128
N_SR = NW // N_DG
S_PER_W = S // N_SR
S_CHUNK = 128
N_CHUNKS = S_PER_W // S_CHUNK
L = 16


def _sc_body(x_hbm, pe_hbm, out_hbm, xt, pet, xsem, psem, osem):
    wid = lax.axis_index("s") * 2 + lax.axis_index("c")
    dg = wid % N_DG
    d0 = dg * D_PER_W
    pe_d0 = (dg // 2) * PE_DW
    d_half = (dg % 2) * D_PER_W
    s_base = (wid // N_DG) * S_PER_W
    iota = lax.iota(jnp.int32, L)

    def pe_copy(c):
        s0 = s_base + c * S_CHUNK
        return pltpu.make_async_copy(
            pe_hbm.at[pl.ds(s0, S_CHUNK), pl.ds(pe_d0, PE_DW)],
            pet.at[c % 2], psem.at[c % 2])

    def x_copy(c, p):
        s0 = s_base + c * S_CHUNK
        return pltpu.make_async_copy(
            x_hbm.at[:, pl.ds(d0, D_PER_W), pl.ds(s0, S_CHUNK)],
            xt.at[p], xsem.at[p])

    def out_copy(c, p):
        s0 = s_base + c * S_CHUNK
        return pltpu.make_async_copy(
            xt.at[p],
            out_hbm.at[:, pl.ds(d0, D_PER_W), pl.ds(s0, S_CHUNK)],
            osem.at[p])

    pe_copy(0).start()
    x_copy(0, 0).start()

    def chunk_body(c, carry):
        p = c % 2
        q = 1 - p

        @pl.when(c >= 1)
        def _():
            out_copy(c - 1, q).wait()

        @pl.when(c + 1 < N_CHUNKS)
        def _():
            pe_copy(c + 1).start()
            x_copy(c + 1, q).start()

        pe_copy(c).wait()
        x_copy(c, p).wait()

        def d_body(d, carry2):
            d_idx = jnp.zeros((L,), jnp.int32) + (d_half + d)
            pvs = [plsc.load_gather(pet.at[p], [sj * L + iota, d_idx])
                   for sj in range(S_CHUNK // L)]
            for b in range(B):
                for sj in range(S_CHUNK // L):
                    sl = xt.at[p, b, d, pl.ds(sj * L, L)]
                    sl[...] = sl[...] + pvs[sj]
            return carry2

        lax.fori_loop(0, D_PER_W, d_body, 0)

        out_copy(c, p).start()
        return carry

    lax.fori_loop(0, N_CHUNKS, chunk_body, 0)

    out_copy(N_CHUNKS - 1, (N_CHUNKS - 1) % 2).wait()


def kernel(x, pe_table):
    mesh = plsc.VectorSubcoreMesh(core_axis_name="c", subcore_axis_name="s")
    k = functools.partial(
        pl.kernel,
        mesh=mesh,
        out_type=jax.ShapeDtypeStruct((B, D, S), jnp.float32),
        scratch_types=[
            pltpu.VMEM((2, B, D_PER_W, S_CHUNK), jnp.float32),
            pltpu.VMEM((2, S_CHUNK, PE_DW), jnp.float32),
            pltpu.SemaphoreType.DMA((2,)),
            pltpu.SemaphoreType.DMA((2,)),
            pltpu.SemaphoreType.DMA((2,)),
        ],
        compiler_params=pltpu.CompilerParams(needs_layout_passes=False),
    )(_sc_body)
    return k(x, pe_table)


# speedup vs baseline: 2.1617x; 2.1617x over previous
import functools

import jax
import jax.numpy as jnp
from jax import lax
from jax.experimental import pallas as pl
from jax.experimental.pallas import tpu as pltpu
from jax.experimental.pallas import tpu_sc as plsc

B, D, S = 4, 1024, 8192
NW = 32
N_DG = 16
D_PER_W = D // N_DG
PE_DW = 128
N_SR = NW // N_DG
S_PER_W = S // N_SR
S_CHUNK = 128
N_CHUNKS = S_PER_W // S_CHUNK
L = 16


def _sc_body(x_hbm, pe_hbm, out_hbm, xt, pet, xsem, psem, osem):
    wid = lax.axis_index("s") * 2 + lax.axis_index("c")
    dg = wid % N_DG
    d0 = dg * D_PER_W
    pe_d0 = (dg // 2) * PE_DW
    d_half = (dg % 2) * D_PER_W
    s_base = (wid // N_DG) * S_PER_W
    iota = lax.iota(jnp.int32, L)

    def pe_copy(c):
        s0 = s_base + c * S_CHUNK
        return pltpu.make_async_copy(
            pe_hbm.at[pl.ds(s0, S_CHUNK), pl.ds(pe_d0, PE_DW)],
            pet.at[c % 2], psem.at[c % 2])

    def x_copy(c, p):
        s0 = s_base + c * S_CHUNK
        return pltpu.make_async_copy(
            x_hbm.at[:, pl.ds(d0, D_PER_W), pl.ds(s0, S_CHUNK)],
            xt.at[p], xsem.at[p])

    def out_copy(c, p):
        s0 = s_base + c * S_CHUNK
        return pltpu.make_async_copy(
            xt.at[p],
            out_hbm.at[:, pl.ds(d0, D_PER_W), pl.ds(s0, S_CHUNK)],
            osem.at[p])

    pe_copy(0).start()
    x_copy(0, 0).start()

    def chunk_body(c, carry):
        p = c % 2
        q = 1 - p

        @pl.when(c >= 1)
        def _():
            out_copy(c - 1, q).wait()

        @pl.when(c + 1 < N_CHUNKS)
        def _():
            pe_copy(c + 1).start()
            x_copy(c + 1, q).start()

        pe_copy(c).wait()
        x_copy(c, p).wait()

        @plsc.parallel_loop(0, D_PER_W, 1, unroll=2)
        def d_body(d):
            d_idx = jnp.zeros((L,), jnp.int32) + (d_half + d)
            pvs = [plsc.load_gather(pet.at[p], [sj * L + iota, d_idx])
                   for sj in range(S_CHUNK // L)]
            for b in range(B):
                for sj in range(S_CHUNK // L):
                    plsc.addupdate(xt.at[p, b, d, pl.ds(sj * L, L)], pvs[sj])

        out_copy(c, p).start()
        return carry

    lax.fori_loop(0, N_CHUNKS, chunk_body, 0)

    out_copy(N_CHUNKS - 1, (N_CHUNKS - 1) % 2).wait()


def kernel(x, pe_table):
    mesh = plsc.VectorSubcoreMesh(core_axis_name="c", subcore_axis_name="s")
    k = functools.partial(
        pl.kernel,
        mesh=mesh,
        out_type=jax.ShapeDtypeStruct((B, D, S), jnp.float32),
        scratch_types=[
            pltpu.VMEM((2, B, D_PER_W, S_CHUNK), jnp.float32),
            pltpu.VMEM((2, S_CHUNK, PE_DW), jnp.float32),
            pltpu.SemaphoreType.DMA((2,)),
            pltpu.SemaphoreType.DMA((2,)),
            pltpu.SemaphoreType.DMA((2,)),
        ],
        compiler_params=pltpu.CompilerParams(needs_layout_passes=False),
    )(_sc_body)
    return k(x, pe_table)


# speedup vs baseline: 5.6483x; 2.6129x over previous
import jax
import jax.numpy as jnp
from jax.experimental import pallas as pl


def _body(x_ref, pe_ref, out_ref):
    pe_t = jnp.transpose(pe_ref[...], (1, 0))
    out_ref[...] = x_ref[...] + pe_t[None, :, :]


def kernel(x, pe_table):
    b, d, s = x.shape
    s_blk = 512
    b_blk = 4
    grid = (s // s_blk, b // b_blk)
    return pl.pallas_call(
        _body,
        grid=grid,
        in_specs=[
            pl.BlockSpec((b_blk, d, s_blk), lambda si, bi: (bi, 0, si)),
            pl.BlockSpec((s_blk, d), lambda si, bi: (si, 0)),
        ],
        out_specs=pl.BlockSpec((b_blk, d, s_blk), lambda si, bi: (bi, 0, si)),
        out_shape=jax.ShapeDtypeStruct((b, d, s), x.dtype),
    )(x, pe_table)
